# initial kernel scaffold (unmeasured)
import jax
import jax.numpy as jnp
from jax import lax
from jax.experimental import pallas as pl
from jax.experimental.pallas import tpu as pltpu

N_DEV = 8


def _gelu(y):
    c = 0.7978845608028654
    return 0.5 * y * (1.0 + jnp.tanh(c * (y + 0.044715 * y * y * y)))


def kernel(x, w_mat):
    x = x.astype(jnp.bfloat16)
    w = w_mat.astype(jnp.bfloat16)
    m, _ = x.shape
    _, n = w.shape
    m_per = m // N_DEV

    def body(x_ref, w_ref, out_ref, comm_ref, send_sems, recv_sems, credit_sem):
        my = lax.axis_index("i")
        left = lax.rem(my - 1 + N_DEV, N_DEV)
        right = lax.rem(my + 1, N_DEV)

        barrier_sem = pltpu.get_barrier_semaphore()
        for nbr in (left, right):
            pl.semaphore_signal(
                barrier_sem, inc=1,
                device_id=(nbr,), device_id_type=pl.DeviceIdType.MESH,
            )
        pl.semaphore_wait(barrier_sem, 2)

        def partial(chunk):
            rows = x_ref[pl.ds(chunk * m_per, m_per), :]
            return jnp.dot(rows, w_ref[:, :], preferred_element_type=jnp.float32)

        c0 = lax.rem(my - 1 + N_DEV, N_DEV)
        comm_ref[0, :, :] = partial(c0).astype(jnp.bfloat16)

        for s in range(N_DEV - 1):
            send_slot = s % 2
            recv_slot = (s + 1) % 2
            rdma = pltpu.make_async_remote_copy(
                src_ref=comm_ref.at[send_slot],
                dst_ref=comm_ref.at[recv_slot],
                send_sem=send_sems.at[send_slot],
                recv_sem=recv_sems.at[recv_slot],
                device_id=(right,),
                device_id_type=pl.DeviceIdType.MESH,
            )
            if s >= 1:
                pl.semaphore_wait(credit_sem, 1)
            rdma.start()
            rdma.wait()
            if s < N_DEV - 2:
                pl.semaphore_signal(
                    credit_sem, inc=1,
                    device_id=(left,), device_id_type=pl.DeviceIdType.MESH,
                )
            c = lax.rem(my - s - 2 + 2 * N_DEV, N_DEV)
            acc = comm_ref[recv_slot, :, :].astype(jnp.float32) + partial(c)
            if s < N_DEV - 2:
                comm_ref[recv_slot, :, :] = acc.astype(jnp.bfloat16)
            else:
                out_ref[:, :] = _gelu(acc)

    return pl.pallas_call(
        body,
        out_shape=jax.ShapeDtypeStruct((m_per, n), jnp.float32),
        in_specs=[
            pl.BlockSpec(memory_space=pltpu.VMEM),
            pl.BlockSpec(memory_space=pltpu.VMEM),
        ],
        out_specs=pl.BlockSpec(memory_space=pltpu.VMEM),
        scratch_shapes=[
            pltpu.VMEM((2, m_per, n), jnp.bfloat16),
            pltpu.SemaphoreType.DMA((2,)),
            pltpu.SemaphoreType.DMA((2,)),
            pltpu.SemaphoreType.REGULAR,
        ],
        compiler_params=pltpu.CompilerParams(collective_id=0),
    )(x, w)


# baseline (device time: 728762 ns/iter reference)
import jax
import jax.numpy as jnp
from jax import lax
from jax.experimental import pallas as pl
from jax.experimental.pallas import tpu as pltpu

N_DEV = 8


def _gelu(y):
    c = 0.7978845608028654
    return 0.5 * y * (1.0 + jnp.tanh(c * (y + 0.044715 * y * y * y)))


def kernel(x, w_mat):
    x = x.astype(jnp.bfloat16)
    w = w_mat.astype(jnp.bfloat16)
    m, _ = x.shape
    _, n = w.shape
    m_per = m // N_DEV

    def body(x_ref, w_ref, out_ref, comm_ref, send_sems, recv_sems, credit_sem):
        my = lax.axis_index("i")
        left = lax.rem(my - 1 + N_DEV, N_DEV)
        right = lax.rem(my + 1, N_DEV)

        barrier_sem = pltpu.get_barrier_semaphore()
        for nbr in (left, right):
            pl.semaphore_signal(
                barrier_sem, inc=1,
                device_id=(nbr,), device_id_type=pl.DeviceIdType.MESH,
            )
        pl.semaphore_wait(barrier_sem, 2)

        def partial(chunk):
            rows = x_ref[pl.ds(chunk * m_per, m_per), :]
            return jnp.dot(rows, w_ref[:, :], preferred_element_type=jnp.float32)

        c0 = lax.rem(my - 1 + N_DEV, N_DEV)
        comm_ref[0, :, :] = partial(c0).astype(jnp.bfloat16)

        for s in range(N_DEV - 1):
            send_slot = s % 2
            recv_slot = (s + 1) % 2
            rdma = pltpu.make_async_remote_copy(
                src_ref=comm_ref.at[send_slot],
                dst_ref=comm_ref.at[recv_slot],
                send_sem=send_sems.at[send_slot],
                recv_sem=recv_sems.at[recv_slot],
                device_id=(right,),
                device_id_type=pl.DeviceIdType.MESH,
            )
            if s >= 1:
                pl.semaphore_wait(credit_sem, 1)
            rdma.start()
            rdma.wait()
            if s < N_DEV - 2:
                pl.semaphore_signal(
                    credit_sem, inc=1,
                    device_id=(left,), device_id_type=pl.DeviceIdType.MESH,
                )
            c = lax.rem(my - s - 2 + 2 * N_DEV, N_DEV)
            acc = comm_ref[recv_slot, :, :].astype(jnp.float32) + partial(c)
            if s < N_DEV - 2:
                comm_ref[recv_slot, :, :] = acc.astype(jnp.bfloat16)
            else:
                out_ref[:, :] = _gelu(acc)

    return pl.pallas_call(
        body,
        out_shape=jax.ShapeDtypeStruct((m_per, n), jnp.float32),
        in_specs=[
            pl.BlockSpec(memory_space=pltpu.VMEM),
            pl.BlockSpec(memory_space=pltpu.VMEM),
        ],
        out_specs=pl.BlockSpec(memory_space=pltpu.VMEM),
        scratch_shapes=[
            pltpu.VMEM((2, m_per, n), jnp.bfloat16),
            pltpu.SemaphoreType.DMA((2,)),
            pltpu.SemaphoreType.DMA((2,)),
            pltpu.SemaphoreType.REGULAR,
        ],
        compiler_params=pltpu.CompilerParams(
            collective_id=0,
            vmem_limit_bytes=128 * 1024 * 1024,
        ),
    )(x, w)


# device time: 391572 ns/iter; 1.8611x vs baseline; 1.8611x over previous
import jax
import jax.numpy as jnp
from jax import lax
from jax.experimental import pallas as pl
from jax.experimental.pallas import tpu as pltpu

N_DEV = 8


def _gelu(y):
    c = 0.7978845608028654
    return 0.5 * y * (1.0 + jnp.tanh(c * (y + 0.044715 * y * y * y)))


def kernel(x, w_mat):
    x = x.astype(jnp.bfloat16)
    w = w_mat.astype(jnp.bfloat16)
    m, _ = x.shape
    _, n = w.shape
    m_per = m // N_DEV
    n2 = n // 2

    def body(x_ref, w_ref, out_ref, comm_r, comm_l,
             send_r, recv_r, send_l, recv_l, credit_r, credit_l):
        my = lax.axis_index("i")
        left = lax.rem(my - 1 + N_DEV, N_DEV)
        right = lax.rem(my + 1, N_DEV)

        barrier_sem = pltpu.get_barrier_semaphore()
        for nbr in (left, right):
            pl.semaphore_signal(
                barrier_sem, inc=1,
                device_id=(nbr,), device_id_type=pl.DeviceIdType.MESH,
            )
        pl.semaphore_wait(barrier_sem, 2)

        def partial_r(chunk):
            rows = x_ref[pl.ds(chunk * m_per, m_per), :]
            return jnp.dot(rows, w_ref[:, 0:n2],
                           preferred_element_type=jnp.float32)

        def partial_l(chunk):
            rows = x_ref[pl.ds(chunk * m_per, m_per), :]
            return jnp.dot(rows, w_ref[:, n2:n],
                           preferred_element_type=jnp.float32)

        comm_r[0, :, :] = partial_r(lax.rem(my - 1 + N_DEV, N_DEV)).astype(jnp.bfloat16)
        comm_l[0, :, :] = partial_l(lax.rem(my + 1, N_DEV)).astype(jnp.bfloat16)

        for s in range(N_DEV - 1):
            send_slot = s % 2
            recv_slot = (s + 1) % 2
            rdma_r = pltpu.make_async_remote_copy(
                src_ref=comm_r.at[send_slot],
                dst_ref=comm_r.at[recv_slot],
                send_sem=send_r.at[send_slot],
                recv_sem=recv_r.at[recv_slot],
                device_id=(right,),
                device_id_type=pl.DeviceIdType.MESH,
            )
            rdma_l = pltpu.make_async_remote_copy(
                src_ref=comm_l.at[send_slot],
                dst_ref=comm_l.at[recv_slot],
                send_sem=send_l.at[send_slot],
                recv_sem=recv_l.at[recv_slot],
                device_id=(left,),
                device_id_type=pl.DeviceIdType.MESH,
            )
            if s >= 1:
                pl.semaphore_wait(credit_r, 1)
                pl.semaphore_wait(credit_l, 1)
            rdma_r.start()
            rdma_l.start()
            c_r = lax.rem(my - s - 2 + 2 * N_DEV, N_DEV)
            c_l = lax.rem(my + s + 2, N_DEV)
            p_r = partial_r(c_r)
            p_l = partial_l(c_l)
            rdma_r.wait()
            rdma_l.wait()
            if s < N_DEV - 2:
                pl.semaphore_signal(
                    credit_r, inc=1,
                    device_id=(left,), device_id_type=pl.DeviceIdType.MESH,
                )
                pl.semaphore_signal(
                    credit_l, inc=1,
                    device_id=(right,), device_id_type=pl.DeviceIdType.MESH,
                )
            acc_r = comm_r[recv_slot, :, :].astype(jnp.float32) + p_r
            acc_l = comm_l[recv_slot, :, :].astype(jnp.float32) + p_l
            if s < N_DEV - 2:
                comm_r[recv_slot, :, :] = acc_r.astype(jnp.bfloat16)
                comm_l[recv_slot, :, :] = acc_l.astype(jnp.bfloat16)
            else:
                out_ref[:, 0:n2] = _gelu(acc_r)
                out_ref[:, n2:n] = _gelu(acc_l)

    return pl.pallas_call(
        body,
        out_shape=jax.ShapeDtypeStruct((m_per, n), jnp.float32),
        in_specs=[
            pl.BlockSpec(memory_space=pltpu.VMEM),
            pl.BlockSpec(memory_space=pltpu.VMEM),
        ],
        out_specs=pl.BlockSpec(memory_space=pltpu.VMEM),
        scratch_shapes=[
            pltpu.VMEM((2, m_per, n2), jnp.bfloat16),
            pltpu.VMEM((2, m_per, n2), jnp.bfloat16),
            pltpu.SemaphoreType.DMA((2,)),
            pltpu.SemaphoreType.DMA((2,)),
            pltpu.SemaphoreType.DMA((2,)),
            pltpu.SemaphoreType.DMA((2,)),
            pltpu.SemaphoreType.REGULAR,
            pltpu.SemaphoreType.REGULAR,
        ],
        compiler_params=pltpu.CompilerParams(
            collective_id=0,
            vmem_limit_bytes=128 * 1024 * 1024,
        ),
    )(x, w)


# device time: 369714 ns/iter; 1.9712x vs baseline; 1.0591x over previous
import jax
import jax.numpy as jnp
from jax import lax
from jax.experimental import pallas as pl
from jax.experimental.pallas import tpu as pltpu

N_DEV = 8
N_FLOWS = 4


def _gelu(y):
    c = 0.7978845608028654
    return 0.5 * y * (1.0 + jnp.tanh(c * (y + 0.044715 * y * y * y)))


def kernel(x, w_mat):
    x = x.astype(jnp.bfloat16)
    w = w_mat.astype(jnp.bfloat16)
    m, _ = x.shape
    _, n = w.shape
    m_per = m // N_DEV
    npan = n // N_FLOWS

    def body(x_ref, w_ref, out_ref, comm, send_sems, recv_sems, credit_sems):
        my = lax.axis_index("i")
        left = lax.rem(my - 1 + N_DEV, N_DEV)
        right = lax.rem(my + 1, N_DEV)

        barrier_sem = pltpu.get_barrier_semaphore()
        for nbr in (left, right):
            pl.semaphore_signal(
                barrier_sem, inc=1,
                device_id=(nbr,), device_id_type=pl.DeviceIdType.MESH,
            )
        pl.semaphore_wait(barrier_sem, 2)

        def rightward(f):
            return f < 2

        def partial(chunk, f):
            rows = x_ref[pl.ds(chunk * m_per, m_per), :]
            return jnp.dot(rows, w_ref[:, f * npan:(f + 1) * npan],
                           preferred_element_type=jnp.float32)

        def arriving_chunk(f, s):
            if rightward(f):
                return lax.rem(my - s - 2 + 2 * N_DEV, N_DEV)
            return lax.rem(my + s + 2, N_DEV)

        def make_rdma(f, s):
            send_slot = s % 2
            recv_slot = (s + 1) % 2
            dst = right if rightward(f) else left
            return pltpu.make_async_remote_copy(
                src_ref=comm.at[f, send_slot],
                dst_ref=comm.at[f, recv_slot],
                send_sem=send_sems.at[f, send_slot],
                recv_sem=recv_sems.at[f, recv_slot],
                device_id=(dst,),
                device_id_type=pl.DeviceIdType.MESH,
            )

        order = (0, 2, 1, 3)

        seed_r = lax.rem(my - 1 + N_DEV, N_DEV)
        seed_l = lax.rem(my + 1, N_DEV)
        for f in order:
            seed = seed_r if rightward(f) else seed_l
            comm[f, 0, :, :] = partial(seed, f).astype(jnp.bfloat16)
        for f in order:
            make_rdma(f, 0).start()

        for s in range(N_DEV - 1):
            recv_slot = (s + 1) % 2
            for f in order:
                upstream = left if rightward(f) else right
                p = partial(arriving_chunk(f, s), f)
                d = make_rdma(f, s)
                d.wait()
                if s < N_DEV - 2:
                    pl.semaphore_signal(
                        credit_sems.at[f], inc=1,
                        device_id=(upstream,),
                        device_id_type=pl.DeviceIdType.MESH,
                    )
                acc = comm[f, recv_slot, :, :].astype(jnp.float32) + p
                if s < N_DEV - 2:
                    comm[f, recv_slot, :, :] = acc.astype(jnp.bfloat16)
                    pl.semaphore_wait(credit_sems.at[f], 1)
                    make_rdma(f, s + 1).start()
                else:
                    out_ref[:, f * npan:(f + 1) * npan] = _gelu(acc)

    return pl.pallas_call(
        body,
        out_shape=jax.ShapeDtypeStruct((m_per, n), jnp.float32),
        in_specs=[
            pl.BlockSpec(memory_space=pltpu.VMEM),
            pl.BlockSpec(memory_space=pltpu.VMEM),
        ],
        out_specs=pl.BlockSpec(memory_space=pltpu.VMEM),
        scratch_shapes=[
            pltpu.VMEM((N_FLOWS, 2, m_per, npan), jnp.bfloat16),
            pltpu.SemaphoreType.DMA((N_FLOWS, 2)),
            pltpu.SemaphoreType.DMA((N_FLOWS, 2)),
            pltpu.SemaphoreType.REGULAR((N_FLOWS,)),
        ],
        compiler_params=pltpu.CompilerParams(
            collective_id=0,
            vmem_limit_bytes=128 * 1024 * 1024,
        ),
    )(x, w)


# device time: 363242 ns/iter; 2.0063x vs baseline; 1.0178x over previous
import jax
import jax.numpy as jnp
from jax import lax
from jax.experimental import pallas as pl
from jax.experimental.pallas import tpu as pltpu

N_DEV = 8
N_FLOWS = 8


def _gelu(y):
    c = 0.7978845608028654
    return 0.5 * y * (1.0 + jnp.tanh(c * (y + 0.044715 * y * y * y)))


def kernel(x, w_mat):
    x = x.astype(jnp.bfloat16)
    w = w_mat.astype(jnp.bfloat16)
    m, _ = x.shape
    _, n = w.shape
    m_per = m // N_DEV
    npan = n // N_FLOWS

    def body(x_ref, w_ref, out_ref, comm, send_sems, recv_sems, credit_sems):
        my = lax.axis_index("i")
        left = lax.rem(my - 1 + N_DEV, N_DEV)
        right = lax.rem(my + 1, N_DEV)

        barrier_sem = pltpu.get_barrier_semaphore()
        for nbr in (left, right):
            pl.semaphore_signal(
                barrier_sem, inc=1,
                device_id=(nbr,), device_id_type=pl.DeviceIdType.MESH,
            )
        pl.semaphore_wait(barrier_sem, 2)

        def rightward(f):
            return f < N_FLOWS // 2

        def partial(chunk, f):
            rows = x_ref[pl.ds(chunk * m_per, m_per), :]
            return jnp.dot(rows, w_ref[:, f * npan:(f + 1) * npan],
                           preferred_element_type=jnp.float32)

        def arriving_chunk(f, s):
            if rightward(f):
                return lax.rem(my - s - 2 + 2 * N_DEV, N_DEV)
            return lax.rem(my + s + 2, N_DEV)

        def make_rdma(f, s):
            send_slot = s % 2
            recv_slot = (s + 1) % 2
            dst = right if rightward(f) else left
            return pltpu.make_async_remote_copy(
                src_ref=comm.at[f, send_slot],
                dst_ref=comm.at[f, recv_slot],
                send_sem=send_sems.at[f, send_slot],
                recv_sem=recv_sems.at[f, recv_slot],
                device_id=(dst,),
                device_id_type=pl.DeviceIdType.MESH,
            )

        order = (0, 4, 1, 5, 2, 6, 3, 7)

        seed_r = lax.rem(my - 1 + N_DEV, N_DEV)
        seed_l = lax.rem(my + 1, N_DEV)
        for f in order:
            seed = seed_r if rightward(f) else seed_l
            comm[f, 0, :, :] = partial(seed, f).astype(jnp.bfloat16)
            make_rdma(f, 0).start()

        for s in range(N_DEV - 1):
            recv_slot = (s + 1) % 2
            for f in order:
                upstream = left if rightward(f) else right
                p = partial(arriving_chunk(f, s), f)
                d = make_rdma(f, s)
                d.wait()
                if s < N_DEV - 2:
                    pl.semaphore_signal(
                        credit_sems.at[f], inc=1,
                        device_id=(upstream,),
                        device_id_type=pl.DeviceIdType.MESH,
                    )
                acc = comm[f, recv_slot, :, :].astype(jnp.float32) + p
                if s < N_DEV - 2:
                    comm[f, recv_slot, :, :] = acc.astype(jnp.bfloat16)
                    pl.semaphore_wait(credit_sems.at[f], 1)
                    make_rdma(f, s + 1).start()
                else:
                    out_ref[:, f * npan:(f + 1) * npan] = _gelu(acc)

    return pl.pallas_call(
        body,
        out_shape=jax.ShapeDtypeStruct((m_per, n), jnp.float32),
        in_specs=[
            pl.BlockSpec(memory_space=pltpu.VMEM),
            pl.BlockSpec(memory_space=pltpu.VMEM),
        ],
        out_specs=pl.BlockSpec(memory_space=pltpu.VMEM),
        scratch_shapes=[
            pltpu.VMEM((N_FLOWS, 2, m_per, npan), jnp.bfloat16),
            pltpu.SemaphoreType.DMA((N_FLOWS, 2)),
            pltpu.SemaphoreType.DMA((N_FLOWS, 2)),
            pltpu.SemaphoreType.REGULAR((N_FLOWS,)),
        ],
        compiler_params=pltpu.CompilerParams(
            collective_id=0,
            vmem_limit_bytes=128 * 1024 * 1024,
        ),
    )(x, w)


# device time: 362204 ns/iter; 2.0120x vs baseline; 1.0029x over previous
import jax
import jax.numpy as jnp
from jax import lax
from jax.experimental import pallas as pl
from jax.experimental.pallas import tpu as pltpu

N_DEV = 8
N_FLOWS = 8


def _gelu(y):
    c = 0.7978845608028654
    return 0.5 * y * (1.0 + jnp.tanh(c * (y + 0.044715 * y * y * y)))


def kernel(x, w_mat):
    x = x.astype(jnp.bfloat16)
    w = w_mat.astype(jnp.bfloat16)
    m, _ = x.shape
    _, n = w.shape
    m_per = m // N_DEV
    npan = n // N_FLOWS

    RING = (0, 4, 7, 3, 2, 6, 5, 1)
    RANK = (0, 7, 4, 3, 1, 6, 5, 2)

    def body(x_ref, w_ref, out_ref, comm, send_sems, recv_sems, credit_sems):
        my = lax.axis_index("i")

        def lut(table, idx):
            val = jnp.int32(table[0])
            for j in range(1, N_DEV):
                val = jnp.where(idx == j, jnp.int32(table[j]), val)
            return val

        rank = lut(RANK, my)
        right = lut(RING, lax.rem(rank + 1, N_DEV))
        left = lut(RING, lax.rem(rank - 1 + N_DEV, N_DEV))

        barrier_sem = pltpu.get_barrier_semaphore()
        for nbr in (left, right):
            pl.semaphore_signal(
                barrier_sem, inc=1,
                device_id=(nbr,), device_id_type=pl.DeviceIdType.MESH,
            )
        pl.semaphore_wait(barrier_sem, 2)

        def rightward(f):
            return f < N_FLOWS // 2

        def partial(chunk, f):
            rows = x_ref[pl.ds(chunk * m_per, m_per), :]
            return jnp.dot(rows, w_ref[:, f * npan:(f + 1) * npan],
                           preferred_element_type=jnp.float32)

        def arriving_chunk(f, s):
            if rightward(f):
                return lut(RING, lax.rem(rank - s - 2 + 2 * N_DEV, N_DEV))
            return lut(RING, lax.rem(rank + s + 2, N_DEV))

        def make_rdma(f, s):
            send_slot = s % 2
            recv_slot = (s + 1) % 2
            dst = right if rightward(f) else left
            return pltpu.make_async_remote_copy(
                src_ref=comm.at[f, send_slot],
                dst_ref=comm.at[f, recv_slot],
                send_sem=send_sems.at[f, send_slot],
                recv_sem=recv_sems.at[f, recv_slot],
                device_id=(dst,),
                device_id_type=pl.DeviceIdType.MESH,
            )

        order = (0, 4, 1, 5, 2, 6, 3, 7)

        seed_r = lut(RING, lax.rem(rank - 1 + N_DEV, N_DEV))
        seed_l = lut(RING, lax.rem(rank + 1, N_DEV))
        for f in order:
            seed = seed_r if rightward(f) else seed_l
            comm[f, 0, :, :] = partial(seed, f).astype(jnp.bfloat16)
            make_rdma(f, 0).start()

        for s in range(N_DEV - 1):
            recv_slot = (s + 1) % 2
            for f in order:
                upstream = left if rightward(f) else right
                p = partial(arriving_chunk(f, s), f)
                d = make_rdma(f, s)
                d.wait()
                if s < N_DEV - 2:
                    pl.semaphore_signal(
                        credit_sems.at[f], inc=1,
                        device_id=(upstream,),
                        device_id_type=pl.DeviceIdType.MESH,
                    )
                acc = comm[f, recv_slot, :, :].astype(jnp.float32) + p
                if s < N_DEV - 2:
                    comm[f, recv_slot, :, :] = acc.astype(jnp.bfloat16)
                    pl.semaphore_wait(credit_sems.at[f], 1)
                    make_rdma(f, s + 1).start()
                else:
                    out_ref[:, f * npan:(f + 1) * npan] = _gelu(acc)

    return pl.pallas_call(
        body,
        out_shape=jax.ShapeDtypeStruct((m_per, n), jnp.float32),
        in_specs=[
            pl.BlockSpec(memory_space=pltpu.VMEM),
            pl.BlockSpec(memory_space=pltpu.VMEM),
        ],
        out_specs=pl.BlockSpec(memory_space=pltpu.VMEM),
        scratch_shapes=[
            pltpu.VMEM((N_FLOWS, 2, m_per, npan), jnp.bfloat16),
            pltpu.SemaphoreType.DMA((N_FLOWS, 2)),
            pltpu.SemaphoreType.DMA((N_FLOWS, 2)),
            pltpu.SemaphoreType.REGULAR((N_FLOWS,)),
        ],
        compiler_params=pltpu.CompilerParams(
            collective_id=0,
            vmem_limit_bytes=128 * 1024 * 1024,
        ),
    )(x, w)


# device time: 360897 ns/iter; 2.0193x vs baseline; 1.0036x over previous
import jax
import jax.numpy as jnp
from jax import lax
from jax.experimental import pallas as pl
from jax.experimental.pallas import tpu as pltpu

N_DEV = 8
N_FLOWS = 16


def _gelu(y):
    c = 0.7978845608028654
    return 0.5 * y * (1.0 + jnp.tanh(c * (y + 0.044715 * y * y * y)))


def kernel(x, w_mat):
    x = x.astype(jnp.bfloat16)
    w = w_mat.astype(jnp.bfloat16)
    m, _ = x.shape
    _, n = w.shape
    m_per = m // N_DEV
    npan = n // N_FLOWS

    RING = (0, 4, 7, 3, 2, 6, 5, 1)
    RANK = (0, 7, 4, 3, 1, 6, 5, 2)

    def body(x_ref, w_ref, out_ref, comm, send_sems, recv_sems, credit_sems):
        my = lax.axis_index("i")

        def lut(table, idx):
            val = jnp.int32(table[0])
            for j in range(1, N_DEV):
                val = jnp.where(idx == j, jnp.int32(table[j]), val)
            return val

        rank = lut(RANK, my)
        right = lut(RING, lax.rem(rank + 1, N_DEV))
        left = lut(RING, lax.rem(rank - 1 + N_DEV, N_DEV))

        barrier_sem = pltpu.get_barrier_semaphore()
        for nbr in (left, right):
            pl.semaphore_signal(
                barrier_sem, inc=1,
                device_id=(nbr,), device_id_type=pl.DeviceIdType.MESH,
            )
        pl.semaphore_wait(barrier_sem, 2)

        def rightward(f):
            return f < N_FLOWS // 2

        def partial(chunk, f):
            rows = x_ref[pl.ds(chunk * m_per, m_per), :]
            return jnp.dot(rows, w_ref[:, f * npan:(f + 1) * npan],
                           preferred_element_type=jnp.float32)

        def arriving_chunk(f, s):
            if rightward(f):
                return lut(RING, lax.rem(rank - s - 2 + 2 * N_DEV, N_DEV))
            return lut(RING, lax.rem(rank + s + 2, N_DEV))

        def make_rdma(f, s):
            send_slot = s % 2
            recv_slot = (s + 1) % 2
            dst = right if rightward(f) else left
            return pltpu.make_async_remote_copy(
                src_ref=comm.at[f, send_slot],
                dst_ref=comm.at[f, recv_slot],
                send_sem=send_sems.at[f, send_slot],
                recv_sem=recv_sems.at[f, recv_slot],
                device_id=(dst,),
                device_id_type=pl.DeviceIdType.MESH,
            )

        order = tuple(
            f for pair in zip(range(N_FLOWS // 2), range(N_FLOWS // 2, N_FLOWS))
            for f in pair
        )

        seed_r = lut(RING, lax.rem(rank - 1 + N_DEV, N_DEV))
        seed_l = lut(RING, lax.rem(rank + 1, N_DEV))
        for f in order:
            seed = seed_r if rightward(f) else seed_l
            comm[f, 0, :, :] = partial(seed, f).astype(jnp.bfloat16)
            make_rdma(f, 0).start()

        for s in range(N_DEV - 1):
            recv_slot = (s + 1) % 2
            for f in order:
                upstream = left if rightward(f) else right
                p = partial(arriving_chunk(f, s), f)
                d = make_rdma(f, s)
                d.wait()
                if s < N_DEV - 2:
                    pl.semaphore_signal(
                        credit_sems.at[f], inc=1,
                        device_id=(upstream,),
                        device_id_type=pl.DeviceIdType.MESH,
                    )
                acc = comm[f, recv_slot, :, :].astype(jnp.float32) + p
                if s < N_DEV - 2:
                    comm[f, recv_slot, :, :] = acc.astype(jnp.bfloat16)
                    pl.semaphore_wait(credit_sems.at[f], 1)
                    make_rdma(f, s + 1).start()
                else:
                    out_ref[:, f * npan:(f + 1) * npan] = _gelu(acc)

    return pl.pallas_call(
        body,
        out_shape=jax.ShapeDtypeStruct((m_per, n), jnp.float32),
        in_specs=[
            pl.BlockSpec(memory_space=pltpu.VMEM),
            pl.BlockSpec(memory_space=pltpu.VMEM),
        ],
        out_specs=pl.BlockSpec(memory_space=pltpu.VMEM),
        scratch_shapes=[
            pltpu.VMEM((N_FLOWS, 2, m_per, npan), jnp.bfloat16),
            pltpu.SemaphoreType.DMA((N_FLOWS, 2)),
            pltpu.SemaphoreType.DMA((N_FLOWS, 2)),
            pltpu.SemaphoreType.REGULAR((N_FLOWS,)),
        ],
        compiler_params=pltpu.CompilerParams(
            collective_id=0,
            vmem_limit_bytes=128 * 1024 * 1024,
        ),
    )(x, w)
